# baseline (speedup 1.0000x reference)
import functools

import jax
import jax.numpy as jnp
from jax import lax
from jax.experimental import pallas as pl
from jax.experimental.pallas import tpu as pltpu, tpu_sc as plsc

_INFO = plsc.get_sparse_core_info()
_NC, _NS, _L = _INFO.num_cores, _INFO.num_subcores, _INFO.num_lanes
_NW = _NC * _NS

_N = 1048576
_RW = _N // _NW
_C = 2048
_CHUNKS = _RW // _C
_GROUPS = _C // _L


def _body(in_hbm, out_hbm, in_v, out_v):
    wid = lax.axis_index("s") * _NC + lax.axis_index("c")
    iota = lax.iota(jnp.int32, _L)
    zeros = jnp.zeros((_L,), jnp.float32)
    base_in = wid * (_RW * 3)
    base_out = wid * (_RW * 9)

    def chunk(ch, _):
        pltpu.sync_copy(in_hbm.at[pl.ds(base_in + ch * (3 * _C), 3 * _C)], in_v)

        def group(g, carry):
            gi = g * 48 + 3 * iota
            d0 = plsc.load_gather(in_v, [gi])
            d1 = plsc.load_gather(in_v, [gi + 1])
            d2 = plsc.load_gather(in_v, [gi + 2])
            so = g * 144 + 9 * iota
            plsc.store_scatter(out_v, [so], zeros)
            plsc.store_scatter(out_v, [so + 1], -d2)
            plsc.store_scatter(out_v, [so + 2], d1)
            plsc.store_scatter(out_v, [so + 3], d2)
            plsc.store_scatter(out_v, [so + 4], zeros)
            plsc.store_scatter(out_v, [so + 5], -d0)
            plsc.store_scatter(out_v, [so + 6], -d1)
            plsc.store_scatter(out_v, [so + 7], d0)
            plsc.store_scatter(out_v, [so + 8], zeros)
            return carry

        lax.fori_loop(0, _GROUPS, group, 0)
        pltpu.sync_copy(out_v, out_hbm.at[pl.ds(base_out + ch * (9 * _C), 9 * _C)])
        return _

    lax.fori_loop(0, _CHUNKS, chunk, 0)


_skew = functools.partial(
    pl.kernel,
    out_type=jax.ShapeDtypeStruct((_N * 9,), jnp.float32),
    mesh=plsc.VectorSubcoreMesh(core_axis_name="c", subcore_axis_name="s"),
    scratch_types=[
        pltpu.VMEM((3 * _C,), jnp.float32),
        pltpu.VMEM((9 * _C,), jnp.float32),
    ],
    compiler_params=pltpu.CompilerParams(needs_layout_passes=False),
)(_body)


def kernel(dw):
    n = dw.shape[0]
    flat = jnp.reshape(dw, (n * 3,))
    out = _skew(flat)
    return jnp.reshape(out, (n, 3, 3))


# speedup vs baseline: 38.0896x; 38.0896x over previous
import jax
import jax.numpy as jnp
from jax.experimental import pallas as pl

_N = 1048576
_B = 8192


def _body(x_ref, o_ref):
    x = x_ref[...]
    z = jnp.zeros_like(x[0:1])
    d0, d1, d2 = x[0:1], x[1:2], x[2:3]
    o_ref[0] = jnp.concatenate([z, -d2, d1], axis=0)
    o_ref[1] = jnp.concatenate([d2, z, -d0], axis=0)
    o_ref[2] = jnp.concatenate([-d1, d0, z], axis=0)


_call = pl.pallas_call(
    _body,
    out_shape=jax.ShapeDtypeStruct((3, 3, _N), jnp.float32),
    grid=(_N // _B,),
    in_specs=[pl.BlockSpec((3, _B), lambda i: (0, i))],
    out_specs=pl.BlockSpec((3, 3, _B), lambda i: (0, 0, i)),
)


def kernel(dw):
    o = _call(dw.T)
    return o.transpose(2, 0, 1)


# speedup vs baseline: 85.3392x; 2.2405x over previous
import jax
import jax.numpy as jnp
from jax.experimental import pallas as pl

_N = 1048576
_B = 32768


def _body(x_ref, o_ref):
    x = x_ref[...]
    z = jnp.zeros_like(x[0:1])
    d0, d1, d2 = x[0:1], x[1:2], x[2:3]
    o_ref[0] = jnp.concatenate([z, -d2, d1], axis=0)
    o_ref[1] = jnp.concatenate([d2, z, -d0], axis=0)
    o_ref[2] = jnp.concatenate([-d1, d0, z], axis=0)


_call = pl.pallas_call(
    _body,
    out_shape=jax.ShapeDtypeStruct((3, 3, _N), jnp.float32),
    grid=(_N // _B,),
    in_specs=[pl.BlockSpec((3, _B), lambda i: (0, i))],
    out_specs=pl.BlockSpec((3, 3, _B), lambda i: (0, 0, i)),
)


def kernel(dw):
    o = _call(dw.T)
    return o.transpose(2, 0, 1)


# speedup vs baseline: 124.6592x; 1.4607x over previous
import jax
import jax.numpy as jnp
from jax.experimental import pallas as pl

_N = 1048576
_B = 131072


def _body(x_ref, o_ref):
    x = x_ref[...]
    z = jnp.zeros_like(x[0:1])
    d0, d1, d2 = x[0:1], x[1:2], x[2:3]
    o_ref[0] = jnp.concatenate([z, -d2, d1], axis=0)
    o_ref[1] = jnp.concatenate([d2, z, -d0], axis=0)
    o_ref[2] = jnp.concatenate([-d1, d0, z], axis=0)


_call = pl.pallas_call(
    _body,
    out_shape=jax.ShapeDtypeStruct((3, 3, _N), jnp.float32),
    grid=(_N // _B,),
    in_specs=[pl.BlockSpec((3, _B), lambda i: (0, i))],
    out_specs=pl.BlockSpec((3, 3, _B), lambda i: (0, 0, i)),
)


def kernel(dw):
    o = _call(dw.T)
    return o.transpose(2, 0, 1)


# speedup vs baseline: 125.8445x; 1.0095x over previous
import jax
import jax.numpy as jnp
from jax.experimental import pallas as pl

_N = 1048576
_B = 262144


def _body(x_ref, o_ref):
    x = x_ref[...]
    z = jnp.zeros_like(x[0:1])
    d0, d1, d2 = x[0:1], x[1:2], x[2:3]
    o_ref[0] = jnp.concatenate([z, -d2, d1], axis=0)
    o_ref[1] = jnp.concatenate([d2, z, -d0], axis=0)
    o_ref[2] = jnp.concatenate([-d1, d0, z], axis=0)


_call = pl.pallas_call(
    _body,
    out_shape=jax.ShapeDtypeStruct((3, 3, _N), jnp.float32),
    grid=(_N // _B,),
    in_specs=[pl.BlockSpec((3, _B), lambda i: (0, i))],
    out_specs=pl.BlockSpec((3, 3, _B), lambda i: (0, 0, i)),
)


def kernel(dw):
    o = _call(dw.T)
    return o.transpose(2, 0, 1)
